# Initial kernel scaffold; baseline (speedup 1.0000x reference)
import jax
import jax.numpy as jnp
from jax.experimental import pallas as pl


def kernel(ACWA_embeddings, node_1, node_2, node_2_negative):
    raise NotImplementedError("write your pallas kernel here")


# speedup vs baseline: 4.7746x; 4.7746x over previous
import functools

import jax
import jax.numpy as jnp
from jax import lax
from jax.experimental import pallas as pl
from jax.experimental.pallas import tpu as pltpu
from jax.experimental.pallas import tpu_sc as plsc

N_ROWS = 100000
D = 128
B = 200000

NC = 2
NS = 16
NW = NC * NS

C = 112
CH = 56
B_PAD = NW * C * CH
ROWS_PER_W = C * CH


def _sc_score_body(table, n1, n2, n3, pos_out, neg_out,
                   idx1_v, idx2_v, idx3_v, rows_v, pos_s, neg_s, sems):
    wid = lax.axis_index("s") * NC + lax.axis_index("c")
    row0 = wid * CH

    pltpu.sync_copy(n1.at[pl.ds(row0, CH)], idx1_v)
    pltpu.sync_copy(n2.at[pl.ds(row0, CH)], idx2_v)
    pltpu.sync_copy(n3.at[pl.ds(row0, CH)], idx3_v)

    idx_refs = (idx1_v, idx2_v, idx3_v)

    def issue(t, b):
        for k in range(3):
            pltpu.make_async_copy(
                table.at[idx_refs[k].at[t]], rows_v.at[b, k], sems.at[b, k]
            ).start()

    def drain(b):
        for k in range(3):
            pltpu.make_async_copy(
                table.at[idx_refs[k].at[0]], rows_v.at[b, k], sems.at[b, k]
            ).wait()

    issue(0, 0)

    lane = lax.iota(jnp.int32, 16)
    zeros = jnp.zeros((16,), jnp.float32)
    perms = [(lane ^ m).reshape(16, 1) for m in (1, 2, 4, 8)]
    dnums = lax.GatherDimensionNumbers(
        offset_dims=(), collapsed_slice_dims=(0,), start_index_map=(0,))

    def hsum(v):
        for p in perms:
            v = v + lax.gather(v, p, dnums, slice_sizes=(1,),
                               mode=lax.GatherScatterMode.PROMISE_IN_BOUNDS)
        return v

    def chunk_body(t, _):
        b = lax.rem(t, 2)

        @pl.when(t + 1 < CH)
        def _():
            issue(t + 1, 1 - b)

        drain(b)

        def group_body(g, _):
            def edge_body(i, carry):
                accp, accn = carry
                e = g * 16 + i
                vp = rows_v[b, 0, e, pl.ds(0, 16)] * rows_v[b, 1, e, pl.ds(0, 16)]
                vn = rows_v[b, 0, e, pl.ds(0, 16)] * rows_v[b, 2, e, pl.ds(0, 16)]
                for c in range(1, 8):
                    s = rows_v[b, 0, e, pl.ds(c * 16, 16)]
                    vp = vp + s * rows_v[b, 1, e, pl.ds(c * 16, 16)]
                    vn = vn + s * rows_v[b, 2, e, pl.ds(c * 16, 16)]
                m = lane == i
                accp = jnp.where(m, hsum(vp), accp)
                accn = jnp.where(m, hsum(vn), accn)
                return accp, accn

            accp, accn = lax.fori_loop(0, 16, edge_body, (zeros, zeros))
            pos_s[t, pl.ds(g * 16, 16)] = accp
            neg_s[t, pl.ds(g * 16, 16)] = accn
            return 0

        lax.fori_loop(0, C // 16, group_body, 0)
        return 0

    lax.fori_loop(0, CH, chunk_body, 0)

    pltpu.sync_copy(pos_s, pos_out.at[pl.ds(row0, CH)])
    pltpu.sync_copy(neg_s, neg_out.at[pl.ds(row0, CH)])


@jax.jit
def _sc_scores(table, n1, n2, n3):
    mesh = plsc.VectorSubcoreMesh(
        core_axis_name="c", subcore_axis_name="s", num_cores=NC, num_subcores=NS
    )
    f = pl.kernel(
        _sc_score_body,
        out_type=(
            jax.ShapeDtypeStruct((B_PAD // C, C), jnp.float32),
            jax.ShapeDtypeStruct((B_PAD // C, C), jnp.float32),
        ),
        mesh=mesh,
        scratch_types=[
            pltpu.VMEM((CH, C), jnp.int32),
            pltpu.VMEM((CH, C), jnp.int32),
            pltpu.VMEM((CH, C), jnp.int32),
            pltpu.VMEM((2, 3, C, D), jnp.float32),
            pltpu.VMEM((CH, C), jnp.float32),
            pltpu.VMEM((CH, C), jnp.float32),
            pltpu.SemaphoreType.DMA((2, 3)),
        ],
    )
    return f(table, n1, n2, n3)


def _tc_loss_body(ps_ref, ns_ref, loss_ref, psig_ref, nsig_ref):
    p = ps_ref[...]
    n = ns_ref[...]
    psig_ref[...] = jax.nn.sigmoid(p)
    nsig_ref[...] = jax.nn.sigmoid(n)
    rows, cols = p.shape
    flat = (lax.broadcasted_iota(jnp.int32, (rows, cols), 0) * cols
            + lax.broadcasted_iota(jnp.int32, (rows, cols), 1))
    valid = flat < B
    pos_sum = jnp.sum(jnp.where(valid, jax.nn.softplus(-p), 0.0))
    neg_sum = jnp.sum(jnp.where(valid, jax.nn.softplus(n), 0.0))
    loss_ref[...] = ((pos_sum + neg_sum) * (1.0 / B)).reshape(1, 1)


@jax.jit
def _tc_loss(ps, ns):
    return pl.pallas_call(
        _tc_loss_body,
        out_shape=(
            jax.ShapeDtypeStruct((1, 1), jnp.float32),
            jax.ShapeDtypeStruct(ps.shape, jnp.float32),
            jax.ShapeDtypeStruct(ns.shape, jnp.float32),
        ),
    )(ps, ns)


def kernel(ACWA_embeddings, node_1, node_2, node_2_negative):
    pad = B_PAD - B

    def prep(idx):
        return jnp.concatenate(
            [idx, jnp.zeros((pad,), idx.dtype)]).reshape(B_PAD // C, C)

    pos_s, neg_s = _sc_scores(
        ACWA_embeddings, prep(node_1), prep(node_2), prep(node_2_negative))

    loss, psig, nsig = _tc_loss(
        pos_s.reshape(B_PAD // D, D), neg_s.reshape(B_PAD // D, D))

    return (
        loss.reshape(()),
        psig.reshape(B_PAD)[:B],
        nsig.reshape(B_PAD)[:B],
    )


# speedup vs baseline: 5.2449x; 1.0985x over previous
import functools

import jax
import jax.numpy as jnp
from jax import lax
from jax.experimental import pallas as pl
from jax.experimental.pallas import tpu as pltpu
from jax.experimental.pallas import tpu_sc as plsc

N_ROWS = 100000
D = 128
B = 200000

NC = 2
NS = 16
NW = NC * NS

C = 112
CHUNKS = 1792
B_PAD = CHUNKS * C

FAST_C = 0
CHF = 72
CHS = 40


def _sc_score_body(table, n1, n2, n3, pos_out, neg_out,
                   idx1_v, idx2_v, idx3_v, rows_v, pos_s, neg_s, sems):
    c = lax.axis_index("c")
    s = lax.axis_index("s")
    is_fast = c == FAST_C
    nch = jnp.where(is_fast, CHF, CHS)
    row0 = jnp.where(is_fast, s * CHF, 16 * CHF + s * CHS)
    row0 = pl.multiple_of(row0, 8)
    cstart = pl.multiple_of(jnp.minimum(row0, CHUNKS - CHF), 8)
    off = row0 - cstart

    pltpu.sync_copy(n1.at[pl.ds(cstart, CHF)], idx1_v)
    pltpu.sync_copy(n2.at[pl.ds(cstart, CHF)], idx2_v)
    pltpu.sync_copy(n3.at[pl.ds(cstart, CHF)], idx3_v)

    idx_refs = (idx1_v, idx2_v, idx3_v)

    def issue(t, b):
        for k in range(3):
            pltpu.make_async_copy(
                table.at[idx_refs[k].at[off + t]], rows_v.at[b, k], sems.at[b, k]
            ).start()

    def drain(b):
        for k in range(3):
            pltpu.make_async_copy(
                table.at[idx_refs[k].at[0]], rows_v.at[b, k], sems.at[b, k]
            ).wait()

    issue(0, 0)

    lane = lax.iota(jnp.int32, 16)
    zeros = jnp.zeros((16,), jnp.float32)
    perms = [(lane ^ m).reshape(16, 1) for m in (1, 2, 4, 8)]
    dnums = lax.GatherDimensionNumbers(
        offset_dims=(), collapsed_slice_dims=(0,), start_index_map=(0,))

    def hsum(v):
        for p in perms:
            v = v + lax.gather(v, p, dnums, slice_sizes=(1,),
                               mode=lax.GatherScatterMode.PROMISE_IN_BOUNDS)
        return v

    def chunk_body(t, _):
        b = lax.rem(t, 2)

        @pl.when(t + 1 < nch)
        def _():
            issue(t + 1, 1 - b)

        drain(b)

        tm = lax.rem(t, CHS)

        def group_body(g, _):
            def edge_body(i, carry):
                accp, accn = carry
                e = g * 16 + i
                vp = rows_v[b, 0, e, pl.ds(0, 16)] * rows_v[b, 1, e, pl.ds(0, 16)]
                vn = rows_v[b, 0, e, pl.ds(0, 16)] * rows_v[b, 2, e, pl.ds(0, 16)]
                for c in range(1, 8):
                    s = rows_v[b, 0, e, pl.ds(c * 16, 16)]
                    vp = vp + s * rows_v[b, 1, e, pl.ds(c * 16, 16)]
                    vn = vn + s * rows_v[b, 2, e, pl.ds(c * 16, 16)]
                m = lane == i
                accp = jnp.where(m, hsum(vp), accp)
                accn = jnp.where(m, hsum(vn), accn)
                return accp, accn

            accp, accn = lax.fori_loop(0, 16, edge_body, (zeros, zeros))
            pos_s[tm, pl.ds(g * 16, 16)] = accp
            neg_s[tm, pl.ds(g * 16, 16)] = accn
            return 0

        lax.fori_loop(0, C // 16, group_body, 0)

        @pl.when(t == CHS - 1)
        def _():
            pltpu.sync_copy(pos_s, pos_out.at[pl.ds(row0, CHS)])
            pltpu.sync_copy(neg_s, neg_out.at[pl.ds(row0, CHS)])

        return 0

    lax.fori_loop(0, nch, chunk_body, 0)

    @pl.when(is_fast)
    def _():
        pltpu.sync_copy(pos_s.at[pl.ds(0, CHF - CHS)],
                        pos_out.at[pl.ds(row0 + CHS, CHF - CHS)])
        pltpu.sync_copy(neg_s.at[pl.ds(0, CHF - CHS)],
                        neg_out.at[pl.ds(row0 + CHS, CHF - CHS)])


@jax.jit
def _sc_scores(table, n1, n2, n3):
    mesh = plsc.VectorSubcoreMesh(
        core_axis_name="c", subcore_axis_name="s", num_cores=NC, num_subcores=NS
    )
    f = pl.kernel(
        _sc_score_body,
        out_type=(
            jax.ShapeDtypeStruct((B_PAD // C, C), jnp.float32),
            jax.ShapeDtypeStruct((B_PAD // C, C), jnp.float32),
        ),
        mesh=mesh,
        scratch_types=[
            pltpu.VMEM((CHF, C), jnp.int32),
            pltpu.VMEM((CHF, C), jnp.int32),
            pltpu.VMEM((CHF, C), jnp.int32),
            pltpu.VMEM((2, 3, C, D), jnp.float32),
            pltpu.VMEM((CHS, C), jnp.float32),
            pltpu.VMEM((CHS, C), jnp.float32),
            pltpu.SemaphoreType.DMA((2, 3)),
        ],
    )
    return f(table, n1, n2, n3)


def _tc_loss_body(ps_ref, ns_ref, loss_ref, psig_ref, nsig_ref):
    p = ps_ref[...]
    n = ns_ref[...]
    psig_ref[...] = jax.nn.sigmoid(p)
    nsig_ref[...] = jax.nn.sigmoid(n)
    rows, cols = p.shape
    flat = (lax.broadcasted_iota(jnp.int32, (rows, cols), 0) * cols
            + lax.broadcasted_iota(jnp.int32, (rows, cols), 1))
    valid = flat < B
    pos_sum = jnp.sum(jnp.where(valid, jax.nn.softplus(-p), 0.0))
    neg_sum = jnp.sum(jnp.where(valid, jax.nn.softplus(n), 0.0))
    loss_ref[...] = ((pos_sum + neg_sum) * (1.0 / B)).reshape(1, 1)


@jax.jit
def _tc_loss(ps, ns):
    return pl.pallas_call(
        _tc_loss_body,
        out_shape=(
            jax.ShapeDtypeStruct((1, 1), jnp.float32),
            jax.ShapeDtypeStruct(ps.shape, jnp.float32),
            jax.ShapeDtypeStruct(ns.shape, jnp.float32),
        ),
    )(ps, ns)


def kernel(ACWA_embeddings, node_1, node_2, node_2_negative):
    pad = B_PAD - B

    def prep(idx):
        return jnp.concatenate(
            [idx, jnp.zeros((pad,), idx.dtype)]).reshape(B_PAD // C, C)

    pos_s, neg_s = _sc_scores(
        ACWA_embeddings, prep(node_1), prep(node_2), prep(node_2_negative))

    loss, psig, nsig = _tc_loss(
        pos_s.reshape(B_PAD // D, D), neg_s.reshape(B_PAD // D, D))

    return (
        loss.reshape(()),
        psig.reshape(B_PAD)[:B],
        nsig.reshape(B_PAD)[:B],
    )
